# baseline (speedup 1.0000x reference)
import functools

import jax
import jax.numpy as jnp
from jax import lax
from jax.experimental import pallas as pl
from jax.experimental.pallas import tpu as pltpu
from jax.experimental.pallas import tpu_sc as plsc

F32 = jnp.float32

_N = 10000
_E = 160000
_D = 128
_H = 8

_NPAD = 10240
_CH = 128
_TB = 640
_NTB = 1000


def _lrelu(x):
    return jnp.where(x >= 0, x, 0.2 * x)


def _ln(x, g, b):
    mu = jnp.mean(x, axis=-1, keepdims=True)
    var = jnp.mean((x - mu) ** 2, axis=-1, keepdims=True)
    return (x - mu) * jax.lax.rsqrt(var + 1e-5) * g + b


def _edge_body(x_ref, gh_ref, gt_ref,
               wrs_ref, brs_ref, wro_ref, bro_ref,
               wsr_ref, bsr_ref, wor_ref, bor_ref,
               smx_ref, csmx_ref, smh_ref, smt_ref,
               w1_ref, b1_ref, w2_ref, b2_ref,
               g1_ref, c1_ref, g2_ref, c2_ref,
               out_ref, ex_ref, eftsr_ref, eftor_ref):
    x = x_ref[...]
    gh = gh_ref[...]
    gt = gt_ref[...]
    dot = functools.partial(jnp.dot, preferred_element_type=F32)
    bf = jnp.bfloat16
    xb = x.astype(bf)

    th = dot(gh.astype(bf), wrs_ref[...]) + brs_ref[...]
    tt = dot(gt.astype(bf), wro_ref[...]) + bro_ref[...]

    smx = dot(x, smx_ref[...]) + csmx_ref[...]
    smh = dot(gh, smh_ref[...])
    smt = dot(gt, smt_ref[...])

    hs = _lrelu(smx[:, 16:17] + smh[:, 8:9])
    ts = _lrelu(smx[:, 17:18] + smt[:, 8:9])
    m = jnp.maximum(hs, ts)
    ph = jnp.exp(hs - m)
    pt = jnp.exp(ts - m)
    nf = (ph * th + pt * tt) / (ph + pt)

    ex_h = jnp.exp(_lrelu(smh[:, 0:8] + smx[:, 0:8]))
    ex_t = jnp.exp(_lrelu(smt[:, 0:8] + smx[:, 8:16]))
    ex_ref[...] = jnp.concatenate(
        [ex_h, ex_t, jnp.zeros((ex_h.shape[0], 112), F32)], axis=1)

    z = _ln(nf + x, g1_ref[...], c1_ref[...])
    f1 = _lrelu(dot(z.astype(bf), w1_ref[...]) + b1_ref[...])
    f2 = _lrelu(dot(f1.astype(bf), w2_ref[...]) + b2_ref[...])
    out_ref[...] = _ln(f2 + z, g2_ref[...], c2_ref[...])

    eftsr_ref[...] = dot(xb, wsr_ref[...]) + bsr_ref[...]
    eftor_ref[...] = dot(xb, wor_ref[...]) + bor_ref[...]


def _scale_body(ex_ref, sgh_ref, sgt_ref, eftsr_ref, eftor_ref,
                msgh_ref, msgt_ref):
    ex = ex_ref[...]
    wh = jnp.sum(ex[:, 0:8] / (sgh_ref[...][:, 0:8] + 1e-16),
                 axis=1, keepdims=True) * (1.0 / _H)
    wt = jnp.sum(ex[:, 8:16] / (sgt_ref[...][:, 8:16] + 1e-16),
                 axis=1, keepdims=True) * (1.0 / _H)
    msgh_ref[...] = wh * eftsr_ref[...]
    msgt_ref[...] = wt * eftor_ref[...]


def _node_body(aggh_ref, aggt_ref, x_ref,
               w1_ref, b1_ref, w2_ref, b2_ref,
               g1_ref, c1_ref, g2_ref, c2_ref, out_ref):
    dot = functools.partial(jnp.dot, preferred_element_type=F32)
    bf = jnp.bfloat16
    z = _ln(aggh_ref[...] + aggt_ref[...] + x_ref[...],
            g1_ref[...], c1_ref[...])
    f1 = _lrelu(dot(z.astype(bf), w1_ref[...]) + b1_ref[...])
    f2 = _lrelu(dot(f1.astype(bf), w2_ref[...]) + b2_ref[...])
    out_ref[...] = _ln(f2 + z, g2_ref[...], c2_ref[...])


def _sc_gather_body(node_hbm, head_hbm, tail_hbm, gh_out, gt_out,
                    idx_v, rows_v, sem):
    c = lax.axis_index("c")
    s = lax.axis_index("s")
    nchunks = _E // _CH
    lo = nchunks * s // 16
    hi = nchunks * (s + 1) // 16

    def run(idx_hbm, out_hbm):
        def body(r, carry):
            off = r * _CH
            pltpu.sync_copy(idx_hbm.at[pl.ds(off, _CH)], idx_v)
            pltpu.async_copy(node_hbm.at[idx_v], rows_v, sem).wait()
            pltpu.sync_copy(rows_v, out_hbm.at[pl.ds(off, _CH)])
            return carry
        lax.fori_loop(lo, hi, body, 0)

    @pl.when(c == 0)
    def _():
        run(head_hbm, gh_out)

    @pl.when(c == 1)
    def _():
        run(tail_hbm, gt_out)


def _sc_segsum_body(ex_hbm, head_hbm, tail_hbm, sgh_out, sgt_out,
                    idx_v, ex_v, table, sem):
    c = lax.axis_index("c")
    s = lax.axis_index("s")
    nchunks = _E // _CH
    lo = nchunks * s // 16
    hi = nchunks * (s + 1) // 16
    z16 = jnp.zeros((16,), F32)

    def zrow(i, carry):
        for j in range(8):
            ex_v[i, pl.ds(j * 16, 16)] = z16
        return carry
    lax.fori_loop(0, _CH, zrow, 0)

    def zcp(k, carry):
        pltpu.sync_copy(ex_v, table.at[pl.ds(s * 640 + k * _CH, _CH)])
        return carry
    lax.fori_loop(0, 5, zcp, 0)
    plsc.subcore_barrier()

    def scatter_phase(idx_hbm):
        def body(r, carry):
            off = r * _CH
            pltpu.sync_copy(idx_hbm.at[pl.ds(off, _CH)], idx_v)
            pltpu.sync_copy(ex_hbm.at[pl.ds(off, _CH)], ex_v)
            pltpu.sync_copy(ex_v, table.at[idx_v], add=True)
            return carry
        lax.fori_loop(lo, hi, body, 0)

    def gather_phase(idx_hbm, out_hbm):
        def body(r, carry):
            off = r * _CH
            pltpu.sync_copy(idx_hbm.at[pl.ds(off, _CH)], idx_v)
            pltpu.async_copy(table.at[idx_v], ex_v, sem).wait()
            pltpu.sync_copy(ex_v, out_hbm.at[pl.ds(off, _CH)])
            return carry
        lax.fori_loop(lo, hi, body, 0)

    @pl.when(c == 0)
    def _():
        scatter_phase(head_hbm)

    @pl.when(c == 1)
    def _():
        scatter_phase(tail_hbm)

    plsc.subcore_barrier()

    @pl.when(c == 0)
    def _():
        gather_phase(head_hbm, sgh_out)

    @pl.when(c == 1)
    def _():
        gather_phase(tail_hbm, sgt_out)


def _sc_scatter_body(msgh_hbm, msgt_hbm, head_hbm, tail_hbm,
                     aggh_out, aggt_out, idx_v, msg_v, table, sem):
    c = lax.axis_index("c")
    s = lax.axis_index("s")
    nchunks = _E // _CH
    lo = nchunks * s // 16
    hi = nchunks * (s + 1) // 16
    z16 = jnp.zeros((16,), F32)

    def zrow(i, carry):
        for j in range(8):
            msg_v[i, pl.ds(j * 16, 16)] = z16
        return carry
    lax.fori_loop(0, _CH, zrow, 0)

    def zcopy(k, carry):
        pltpu.sync_copy(msg_v, table.at[pl.ds(s * 640 + k * _CH, _CH)])
        return carry
    lax.fori_loop(0, 5, zcopy, 0)
    plsc.subcore_barrier()

    def scatter_phase(idx_hbm, msg_hbm):
        def body(r, carry):
            off = r * _CH
            pltpu.sync_copy(idx_hbm.at[pl.ds(off, _CH)], idx_v)
            pltpu.sync_copy(msg_hbm.at[pl.ds(off, _CH)], msg_v)
            pltpu.sync_copy(msg_v, table.at[idx_v], add=True)
            return carry
        lax.fori_loop(lo, hi, body, 0)

    @pl.when(c == 0)
    def _():
        scatter_phase(head_hbm, msgh_hbm)

    @pl.when(c == 1)
    def _():
        scatter_phase(tail_hbm, msgt_hbm)

    plsc.subcore_barrier()

    def writeout(out_hbm):
        def body(k, carry):
            off = s * 640 + k * _CH
            pltpu.async_copy(table.at[pl.ds(off, _CH)], msg_v, sem).wait()
            pltpu.sync_copy(msg_v, out_hbm.at[pl.ds(off, _CH)])
            return carry
        lax.fori_loop(0, 5, body, 0)

    @pl.when(c == 0)
    def _():
        writeout(aggh_out)

    @pl.when(c == 1)
    def _():
        writeout(aggt_out)


def kernel(node_emb, edge_emb, head_ind, tail_ind, params):
    n, d = node_emb.shape
    e = edge_emb.shape[0]
    assert (n, e, d) == (_N, _E, _D)

    head32 = head_ind.astype(jnp.int32)
    tail32 = tail_ind.astype(jnp.int32)

    w_rs, b_rs = params['W_rs']
    w_ro, b_ro = params['W_ro']
    w_sr, b_sr = params['W_sr']
    w_or, b_or = params['W_or']
    an, bn = params['n2e_att']
    we, be = params['e2n_att']
    an1, an2 = an[:_D], an[_D:]
    wee, wen = we[:_D, 0], we[_D:, 0]

    a_sr = w_sr @ an2
    c_sr = b_sr @ an2 + bn
    a_or = w_or @ an2
    c_or = b_or @ an2 + bn

    smx = jnp.concatenate(
        [a_sr, a_or, wee[:, None], wee[:, None], jnp.zeros((_D, 6), F32)],
        axis=1)
    csmx = jnp.concatenate(
        [c_sr, c_or,
         jnp.asarray([be[0] + b_rs @ wen]),
         jnp.asarray([be[0] + b_ro @ wen]),
         jnp.zeros((6,), F32)])[None, :]
    smh = jnp.concatenate(
        [an1, (w_rs @ wen)[:, None], jnp.zeros((_D, 7), F32)], axis=1)
    smt = jnp.concatenate(
        [an1, (w_ro @ wen)[:, None], jnp.zeros((_D, 7), F32)], axis=1)

    e_l1w, e_l1b = params['e_l1']
    e_l2w, e_l2b = params['e_l2']
    n_l1w, n_l1b = params['n_l1']
    n_l2w, n_l2b = params['n_l2']
    e_g1, e_c1 = params['e_ln1']
    e_g2, e_c2 = params['e_ln2']
    n_g1, n_c1 = params['n_ln1']
    n_g2, n_c2 = params['n_ln2']

    row = lambda v: v[None, :]

    mesh = plsc.VectorSubcoreMesh(core_axis_name="c", subcore_axis_name="s")
    sc_gather = functools.partial(
        pl.kernel,
        out_type=(jax.ShapeDtypeStruct((e, d), F32),
                  jax.ShapeDtypeStruct((e, d), F32)),
        mesh=mesh,
        scratch_types=[pltpu.VMEM((_CH,), jnp.int32),
                       pltpu.VMEM((_CH, d), F32),
                       pltpu.SemaphoreType.DMA],
    )(_sc_gather_body)
    gh, gt = sc_gather(node_emb, head32, tail32)

    grid_e = e // _TB
    full = lambda shp: pl.BlockSpec(shp, lambda i: (0, 0))
    tile = lambda w: pl.BlockSpec((_TB, w), lambda i: (i, 0))
    new_edge, ex, eftsr, eftor = pl.pallas_call(
        _edge_body,
        grid=(grid_e,),
        in_specs=[
            tile(d), tile(d), tile(d),
            full((d, d)), full((1, d)), full((d, d)), full((1, d)),
            full((d, d)), full((1, d)), full((d, d)), full((1, d)),
            full((d, 24)), full((1, 24)), full((d, 16)), full((d, 16)),
            full((d, 4 * d)), full((1, 4 * d)), full((4 * d, d)), full((1, d)),
            full((1, d)), full((1, d)), full((1, d)), full((1, d)),
        ],
        out_specs=[tile(d), tile(d), tile(d), tile(d)],
        out_shape=[jax.ShapeDtypeStruct((e, d), F32),
                   jax.ShapeDtypeStruct((e, d), F32),
                   jax.ShapeDtypeStruct((e, d), F32),
                   jax.ShapeDtypeStruct((e, d), F32)],
    )(edge_emb, gh, gt,
      w_rs.astype(jnp.bfloat16), row(b_rs), w_ro.astype(jnp.bfloat16),
      row(b_ro),
      w_sr.astype(jnp.bfloat16), row(b_sr), w_or.astype(jnp.bfloat16),
      row(b_or),
      smx, csmx, smh, smt,
      e_l1w.astype(jnp.bfloat16), row(e_l1b),
      e_l2w.astype(jnp.bfloat16), row(e_l2b),
      row(e_g1), row(e_c1), row(e_g2), row(e_c2))

    sc_segsum = functools.partial(
        pl.kernel,
        out_type=(jax.ShapeDtypeStruct((e, d), F32),
                  jax.ShapeDtypeStruct((e, d), F32)),
        mesh=mesh,
        scratch_types=[pltpu.VMEM((_CH,), jnp.int32),
                       pltpu.VMEM((_CH, d), F32),
                       pltpu.VMEM_SHARED((_NPAD, d), F32),
                       pltpu.SemaphoreType.DMA],
    )(_sc_segsum_body)
    sgh, sgt = sc_segsum(ex, head32, tail32)

    msgh, msgt = pl.pallas_call(
        _scale_body,
        grid=(grid_e,),
        in_specs=[tile(d), tile(d), tile(d), tile(d), tile(d)],
        out_specs=[tile(d), tile(d)],
        out_shape=[jax.ShapeDtypeStruct((e, d), F32),
                   jax.ShapeDtypeStruct((e, d), F32)],
    )(ex, sgh, sgt, eftsr, eftor)

    sc_scatter = functools.partial(
        pl.kernel,
        out_type=(jax.ShapeDtypeStruct((_NPAD, d), F32),
                  jax.ShapeDtypeStruct((_NPAD, d), F32)),
        mesh=mesh,
        scratch_types=[pltpu.VMEM((_CH,), jnp.int32),
                       pltpu.VMEM((_CH, d), F32),
                       pltpu.VMEM_SHARED((_NPAD, d), F32),
                       pltpu.SemaphoreType.DMA],
    )(_sc_scatter_body)
    aggh, aggt = sc_scatter(msgh, msgt, head32, tail32)

    grid_n = n // _NTB
    ntile = lambda w: pl.BlockSpec((_NTB, w), lambda i: (i, 0))
    new_node = pl.pallas_call(
        _node_body,
        grid=(grid_n,),
        in_specs=[
            ntile(d), ntile(d), ntile(d),
            full((d, 4 * d)), full((1, 4 * d)), full((4 * d, d)), full((1, d)),
            full((1, d)), full((1, d)), full((1, d)), full((1, d)),
        ],
        out_specs=ntile(d),
        out_shape=jax.ShapeDtypeStruct((n, d), F32),
    )(aggh[:n], aggt[:n], node_emb,
      n_l1w.astype(jnp.bfloat16), row(n_l1b),
      n_l2w.astype(jnp.bfloat16), row(n_l2b),
      row(n_g1), row(n_c1), row(n_g2), row(n_c2))

    return (new_node, new_edge)


# speedup vs baseline: 1.0160x; 1.0160x over previous
import functools

import jax
import jax.numpy as jnp
from jax import lax
from jax.experimental import pallas as pl
from jax.experimental.pallas import tpu as pltpu
from jax.experimental.pallas import tpu_sc as plsc

F32 = jnp.float32

_N = 10000
_E = 160000
_D = 128
_H = 8

_NPAD = 10240
_CH = 128
_TB = 640
_NTB = 1000


def _lrelu(x):
    return jnp.where(x >= 0, x, 0.2 * x)


def _ln(x, g, b):
    mu = jnp.mean(x, axis=-1, keepdims=True)
    var = jnp.mean((x - mu) ** 2, axis=-1, keepdims=True)
    return (x - mu) * jax.lax.rsqrt(var + 1e-5) * g + b


def _edge_body(x_ref, gh_ref, gt_ref,
               wrs_ref, brs_ref, wro_ref, bro_ref,
               wsr_ref, bsr_ref, wor_ref, bor_ref,
               smx_ref, csmx_ref, smh_ref, smt_ref,
               w1_ref, b1_ref, w2_ref, b2_ref,
               g1_ref, c1_ref, g2_ref, c2_ref,
               out_ref, ex_ref, eftsr_ref, eftor_ref):
    x = x_ref[...]
    gh = gh_ref[...]
    gt = gt_ref[...]
    dot = functools.partial(jnp.dot, preferred_element_type=F32)

    th = dot(gh, wrs_ref[...]) + brs_ref[...]
    tt = dot(gt, wro_ref[...]) + bro_ref[...]

    smx = dot(x, smx_ref[...]) + csmx_ref[...]
    smh = dot(gh, smh_ref[...])
    smt = dot(gt, smt_ref[...])

    hs = _lrelu(smx[:, 16:17] + smh[:, 8:9])
    ts = _lrelu(smx[:, 17:18] + smt[:, 8:9])
    m = jnp.maximum(hs, ts)
    ph = jnp.exp(hs - m)
    pt = jnp.exp(ts - m)
    nf = (ph * th + pt * tt) / (ph + pt)

    ex_h = jnp.exp(_lrelu(smh[:, 0:8] + smx[:, 0:8]))
    ex_t = jnp.exp(_lrelu(smt[:, 0:8] + smx[:, 8:16]))
    ex_ref[...] = jnp.concatenate(
        [ex_h, ex_t, jnp.zeros((ex_h.shape[0], 112), F32)], axis=1)

    z = _ln(nf + x, g1_ref[...], c1_ref[...])
    f1 = _lrelu(dot(z, w1_ref[...]) + b1_ref[...])
    f2 = _lrelu(dot(f1, w2_ref[...]) + b2_ref[...])
    out_ref[...] = _ln(f2 + z, g2_ref[...], c2_ref[...])

    eftsr_ref[...] = dot(x, wsr_ref[...]) + bsr_ref[...]
    eftor_ref[...] = dot(x, wor_ref[...]) + bor_ref[...]


def _scale_body(ex_ref, sgh_ref, sgt_ref, eftsr_ref, eftor_ref,
                msgh_ref, msgt_ref):
    ex = ex_ref[...]
    wh = jnp.sum(ex[:, 0:8] / (sgh_ref[...][:, 0:8] + 1e-16),
                 axis=1, keepdims=True) * (1.0 / _H)
    wt = jnp.sum(ex[:, 8:16] / (sgt_ref[...][:, 8:16] + 1e-16),
                 axis=1, keepdims=True) * (1.0 / _H)
    msgh_ref[...] = wh * eftsr_ref[...]
    msgt_ref[...] = wt * eftor_ref[...]


def _node_body(aggh_ref, aggt_ref, x_ref,
               w1_ref, b1_ref, w2_ref, b2_ref,
               g1_ref, c1_ref, g2_ref, c2_ref, out_ref):
    dot = functools.partial(jnp.dot, preferred_element_type=F32)
    z = _ln(aggh_ref[...] + aggt_ref[...] + x_ref[...],
            g1_ref[...], c1_ref[...])
    f1 = _lrelu(dot(z, w1_ref[...]) + b1_ref[...])
    f2 = _lrelu(dot(f1, w2_ref[...]) + b2_ref[...])
    out_ref[...] = _ln(f2 + z, g2_ref[...], c2_ref[...])


def _sc_gather_body(node_hbm, head_hbm, tail_hbm, gh_out, gt_out,
                    idx_v, rows_v, sem):
    c = lax.axis_index("c")
    s = lax.axis_index("s")
    nchunks = _E // _CH
    lo = nchunks * s // 16
    hi = nchunks * (s + 1) // 16

    def run(idx_hbm, out_hbm):
        def body(r, carry):
            off = r * _CH
            pltpu.sync_copy(idx_hbm.at[pl.ds(off, _CH)], idx_v)
            pltpu.async_copy(node_hbm.at[idx_v], rows_v, sem).wait()
            pltpu.sync_copy(rows_v, out_hbm.at[pl.ds(off, _CH)])
            return carry
        lax.fori_loop(lo, hi, body, 0)

    @pl.when(c == 0)
    def _():
        run(head_hbm, gh_out)

    @pl.when(c == 1)
    def _():
        run(tail_hbm, gt_out)


def _sc_segsum_body(ex_hbm, head_hbm, tail_hbm, sgh_out, sgt_out,
                    idx_v, ex_v, table, sem):
    c = lax.axis_index("c")
    s = lax.axis_index("s")
    nchunks = _E // _CH
    lo = nchunks * s // 16
    hi = nchunks * (s + 1) // 16
    z16 = jnp.zeros((16,), F32)

    def zrow(i, carry):
        for j in range(8):
            ex_v[i, pl.ds(j * 16, 16)] = z16
        return carry
    lax.fori_loop(0, _CH, zrow, 0)

    def zcp(k, carry):
        pltpu.sync_copy(ex_v, table.at[pl.ds(s * 640 + k * _CH, _CH)])
        return carry
    lax.fori_loop(0, 5, zcp, 0)
    plsc.subcore_barrier()

    def scatter_phase(idx_hbm):
        def body(r, carry):
            off = r * _CH
            pltpu.sync_copy(idx_hbm.at[pl.ds(off, _CH)], idx_v)
            pltpu.sync_copy(ex_hbm.at[pl.ds(off, _CH)], ex_v)
            pltpu.sync_copy(ex_v, table.at[idx_v], add=True)
            return carry
        lax.fori_loop(lo, hi, body, 0)

    def gather_phase(idx_hbm, out_hbm):
        def body(r, carry):
            off = r * _CH
            pltpu.sync_copy(idx_hbm.at[pl.ds(off, _CH)], idx_v)
            pltpu.async_copy(table.at[idx_v], ex_v, sem).wait()
            pltpu.sync_copy(ex_v, out_hbm.at[pl.ds(off, _CH)])
            return carry
        lax.fori_loop(lo, hi, body, 0)

    @pl.when(c == 0)
    def _():
        scatter_phase(head_hbm)

    @pl.when(c == 1)
    def _():
        scatter_phase(tail_hbm)

    plsc.subcore_barrier()

    @pl.when(c == 0)
    def _():
        gather_phase(head_hbm, sgh_out)

    @pl.when(c == 1)
    def _():
        gather_phase(tail_hbm, sgt_out)


def _sc_scatter_body(msgh_hbm, msgt_hbm, head_hbm, tail_hbm,
                     aggh_out, aggt_out, idx_v, msg_v, table, sem):
    c = lax.axis_index("c")
    s = lax.axis_index("s")
    nchunks = _E // _CH
    lo = nchunks * s // 16
    hi = nchunks * (s + 1) // 16
    z16 = jnp.zeros((16,), F32)

    def zrow(i, carry):
        for j in range(8):
            msg_v[i, pl.ds(j * 16, 16)] = z16
        return carry
    lax.fori_loop(0, _CH, zrow, 0)

    def zcopy(k, carry):
        pltpu.sync_copy(msg_v, table.at[pl.ds(s * 640 + k * _CH, _CH)])
        return carry
    lax.fori_loop(0, 5, zcopy, 0)
    plsc.subcore_barrier()

    def scatter_phase(idx_hbm, msg_hbm):
        def body(r, carry):
            off = r * _CH
            pltpu.sync_copy(idx_hbm.at[pl.ds(off, _CH)], idx_v)
            pltpu.sync_copy(msg_hbm.at[pl.ds(off, _CH)], msg_v)
            pltpu.sync_copy(msg_v, table.at[idx_v], add=True)
            return carry
        lax.fori_loop(lo, hi, body, 0)

    @pl.when(c == 0)
    def _():
        scatter_phase(head_hbm, msgh_hbm)

    @pl.when(c == 1)
    def _():
        scatter_phase(tail_hbm, msgt_hbm)

    plsc.subcore_barrier()

    def writeout(out_hbm):
        def body(k, carry):
            off = s * 640 + k * _CH
            pltpu.async_copy(table.at[pl.ds(off, _CH)], msg_v, sem).wait()
            pltpu.sync_copy(msg_v, out_hbm.at[pl.ds(off, _CH)])
            return carry
        lax.fori_loop(0, 5, body, 0)

    @pl.when(c == 0)
    def _():
        writeout(aggh_out)

    @pl.when(c == 1)
    def _():
        writeout(aggt_out)


def kernel(node_emb, edge_emb, head_ind, tail_ind, params):
    n, d = node_emb.shape
    e = edge_emb.shape[0]
    assert (n, e, d) == (_N, _E, _D)

    head32 = head_ind.astype(jnp.int32)
    tail32 = tail_ind.astype(jnp.int32)

    w_rs, b_rs = params['W_rs']
    w_ro, b_ro = params['W_ro']
    w_sr, b_sr = params['W_sr']
    w_or, b_or = params['W_or']
    an, bn = params['n2e_att']
    we, be = params['e2n_att']
    an1, an2 = an[:_D], an[_D:]
    wee, wen = we[:_D, 0], we[_D:, 0]

    a_sr = w_sr @ an2
    c_sr = b_sr @ an2 + bn
    a_or = w_or @ an2
    c_or = b_or @ an2 + bn

    smx = jnp.concatenate(
        [a_sr, a_or, wee[:, None], wee[:, None], jnp.zeros((_D, 6), F32)],
        axis=1)
    csmx = jnp.concatenate(
        [c_sr, c_or,
         jnp.asarray([be[0] + b_rs @ wen]),
         jnp.asarray([be[0] + b_ro @ wen]),
         jnp.zeros((6,), F32)])[None, :]
    smh = jnp.concatenate(
        [an1, (w_rs @ wen)[:, None], jnp.zeros((_D, 7), F32)], axis=1)
    smt = jnp.concatenate(
        [an1, (w_ro @ wen)[:, None], jnp.zeros((_D, 7), F32)], axis=1)

    e_l1w, e_l1b = params['e_l1']
    e_l2w, e_l2b = params['e_l2']
    n_l1w, n_l1b = params['n_l1']
    n_l2w, n_l2b = params['n_l2']
    e_g1, e_c1 = params['e_ln1']
    e_g2, e_c2 = params['e_ln2']
    n_g1, n_c1 = params['n_ln1']
    n_g2, n_c2 = params['n_ln2']

    row = lambda v: v[None, :]

    mesh = plsc.VectorSubcoreMesh(core_axis_name="c", subcore_axis_name="s")
    sc_gather = functools.partial(
        pl.kernel,
        out_type=(jax.ShapeDtypeStruct((e, d), F32),
                  jax.ShapeDtypeStruct((e, d), F32)),
        mesh=mesh,
        scratch_types=[pltpu.VMEM((_CH,), jnp.int32),
                       pltpu.VMEM((_CH, d), F32),
                       pltpu.SemaphoreType.DMA],
    )(_sc_gather_body)
    gh, gt = sc_gather(node_emb, head32, tail32)

    grid_e = e // _TB
    full = lambda shp: pl.BlockSpec(shp, lambda i: (0, 0))
    tile = lambda w: pl.BlockSpec((_TB, w), lambda i: (i, 0))
    new_edge, ex, eftsr, eftor = pl.pallas_call(
        _edge_body,
        grid=(grid_e,),
        in_specs=[
            tile(d), tile(d), tile(d),
            full((d, d)), full((1, d)), full((d, d)), full((1, d)),
            full((d, d)), full((1, d)), full((d, d)), full((1, d)),
            full((d, 24)), full((1, 24)), full((d, 16)), full((d, 16)),
            full((d, 4 * d)), full((1, 4 * d)), full((4 * d, d)), full((1, d)),
            full((1, d)), full((1, d)), full((1, d)), full((1, d)),
        ],
        out_specs=[tile(d), tile(d), tile(d), tile(d)],
        out_shape=[jax.ShapeDtypeStruct((e, d), F32),
                   jax.ShapeDtypeStruct((e, d), F32),
                   jax.ShapeDtypeStruct((e, d), F32),
                   jax.ShapeDtypeStruct((e, d), F32)],
    )(edge_emb, gh, gt,
      w_rs, row(b_rs), w_ro, row(b_ro),
      w_sr, row(b_sr), w_or, row(b_or),
      smx, csmx, smh, smt,
      e_l1w, row(e_l1b), e_l2w, row(e_l2b),
      row(e_g1), row(e_c1), row(e_g2), row(e_c2))

    sc_segsum = functools.partial(
        pl.kernel,
        out_type=(jax.ShapeDtypeStruct((e, d), F32),
                  jax.ShapeDtypeStruct((e, d), F32)),
        mesh=mesh,
        scratch_types=[pltpu.VMEM((_CH,), jnp.int32),
                       pltpu.VMEM((_CH, d), F32),
                       pltpu.VMEM_SHARED((_NPAD, d), F32),
                       pltpu.SemaphoreType.DMA],
    )(_sc_segsum_body)
    sgh, sgt = sc_segsum(ex, head32, tail32)

    msgh, msgt = pl.pallas_call(
        _scale_body,
        grid=(grid_e,),
        in_specs=[tile(d), tile(d), tile(d), tile(d), tile(d)],
        out_specs=[tile(d), tile(d)],
        out_shape=[jax.ShapeDtypeStruct((e, d), F32),
                   jax.ShapeDtypeStruct((e, d), F32)],
    )(ex, sgh, sgt, eftsr, eftor)

    sc_scatter = functools.partial(
        pl.kernel,
        out_type=(jax.ShapeDtypeStruct((_NPAD, d), F32),
                  jax.ShapeDtypeStruct((_NPAD, d), F32)),
        mesh=mesh,
        scratch_types=[pltpu.VMEM((_CH,), jnp.int32),
                       pltpu.VMEM((_CH, d), F32),
                       pltpu.VMEM_SHARED((_NPAD, d), F32),
                       pltpu.SemaphoreType.DMA],
    )(_sc_scatter_body)
    aggh, aggt = sc_scatter(msgh, msgt, head32, tail32)

    grid_n = n // _NTB
    ntile = lambda w: pl.BlockSpec((_NTB, w), lambda i: (i, 0))
    new_node = pl.pallas_call(
        _node_body,
        grid=(grid_n,),
        in_specs=[
            ntile(d), ntile(d), ntile(d),
            full((d, 4 * d)), full((1, 4 * d)), full((4 * d, d)), full((1, d)),
            full((1, d)), full((1, d)), full((1, d)), full((1, d)),
        ],
        out_specs=ntile(d),
        out_shape=jax.ShapeDtypeStruct((n, d), F32),
    )(aggh[:n], aggt[:n], node_emb,
      n_l1w, row(n_l1b), n_l2w, row(n_l2b),
      row(n_g1), row(n_c1), row(n_g2), row(n_c2))

    return (new_node, new_edge)
